# baseline (speedup 1.0000x reference)
import functools

import jax
import jax.numpy as jnp
from jax import lax
from jax.experimental import pallas as pl
from jax.experimental.pallas import tpu as pltpu
from jax.experimental.pallas import tpu_sc as plsc

B, T, D, K = 8, 576, 256, 1024


CHUNKS = (576,)


BPS = 4


def _dist_body(boff, x_ref, cb_ref, idx_ref, loss_ref, cbn_s, caug_s):
    i = pl.program_id(0)

    @pl.when(i == 0)
    def _():
        cb = cb_ref[...]
        cbn_s[...] = cb * -2.0
        c_sq = jnp.sum(cb * cb, axis=1)
        caug_s[...] = jnp.broadcast_to(c_sq[None, :], (8, K))
        loss_ref[0, 0] = 0.0

    cbn = cbn_s[...]
    c_sq8 = caug_s[...]

    for bi in range(BPS):
        xi = x_ref[bi]
        x_sq = jnp.sum(xi * xi, axis=1)
        accv = jnp.full((8, K), jnp.inf, jnp.float32)
        acci = jnp.zeros((8, K), jnp.int32)
        cstart = 0
        for tb in CHUNKS:
            xi_c = xi[cstart:cstart + tb, :]
            dc = lax.dot_general(xi_c, cbn, (((1,), (1,)), ((), ())),
                                 preferred_element_type=jnp.float32)
            for g in range(tb // 8):
                t0 = cstart + g * 8
                blk = (c_sq8 + x_sq[t0:t0 + 8, None]) + dc[g * 8:(g + 1) * 8, :]
                upd = blk < accv
                accv = jnp.minimum(accv, blk)
                acci = jnp.where(upd, t0 // 8, acci)
            cstart += tb

        minv = jnp.min(accv, axis=0)
        tcand = acci * 8 + lax.broadcasted_iota(jnp.int32, (8, K), 0)
        amin = jnp.min(jnp.where(accv == minv[None, :], tcand, T), axis=0)
        idx_ref[bi, 0, :] = amin + (boff + i * BPS + bi) * T
        loss_ref[0, 0] += jnp.sum(minv)


def _distances_argmin(x, codebook, boff, nb):
    return pl.pallas_call(
        functools.partial(_dist_body, boff),
        grid=(nb // BPS,),
        in_specs=[
            pl.BlockSpec((BPS, T, D), lambda i: (i, 0, 0)),
            pl.BlockSpec((K, D), lambda i: (0, 0)),
        ],
        out_specs=[
            pl.BlockSpec((BPS, 1, K), lambda i: (i, 0, 0)),
            pl.BlockSpec(block_shape=(1, 1), index_map=lambda i: (0, 0),
                         memory_space=pltpu.SMEM),
        ],
        out_shape=[
            jax.ShapeDtypeStruct((nb, 1, K), jnp.int32),
            jax.ShapeDtypeStruct((1, 1), jnp.float32),
        ],
        scratch_shapes=[
            pltpu.VMEM((K, D), jnp.float32),
            pltpu.VMEM((8, K), jnp.float32),
        ],
    )(x, codebook)


_NC, _NS = 2, 16
_NW = _NC * _NS
_CHUNK = 128


def _gather_body(bpw, table_hbm, idx_hbm, out_hbm, idx_v, rows_v, sem_g0,
                 sem_g1, sem_w):
    wid = lax.axis_index("s") * _NC + lax.axis_index("c")
    base = wid * bpw
    pltpu.sync_copy(idx_hbm.at[pl.ds(base, bpw)], idx_v)
    sems = [sem_g0, sem_g1]
    gathers = [pltpu.async_copy(
        table_hbm.at[idx_v.at[pl.ds(c * _CHUNK, _CHUNK)]],
        rows_v.at[pl.ds(c * _CHUNK, _CHUNK)],
        sems[c]) for c in range(bpw // _CHUNK)]
    writes = []
    for c, g in enumerate(gathers):
        g.wait()
        writes.append(pltpu.async_copy(
            rows_v.at[pl.ds(c * _CHUNK, _CHUNK)],
            out_hbm.at[pl.ds(base + c * _CHUNK, _CHUNK)],
            sem_w))
    for w in writes:
        w.wait()


@functools.cache
def _gather_rows(nrows):
    bpw = nrows // _NW
    return functools.partial(
        pl.kernel,
        mesh=plsc.VectorSubcoreMesh(core_axis_name="c", subcore_axis_name="s"),
        out_type=jax.ShapeDtypeStruct((nrows, D), jnp.float32),
        scratch_types=[
            pltpu.VMEM((bpw,), jnp.int32),
            pltpu.VMEM((bpw, D), jnp.float32),
            pltpu.SemaphoreType.DMA,
            pltpu.SemaphoreType.DMA,
            pltpu.SemaphoreType.DMA,
        ],
    )(functools.partial(_gather_body, bpw))


def kernel(x, codebook):
    idx4, loss_sum = _distances_argmin(x, codebook, 0, B)
    table = x.reshape(B * T, D)
    out = _gather_rows(B * K)(table, idx4.reshape(B * K)).reshape(B, K, D)
    loss = loss_sum[0, 0] * (2.0 / (B * K * D))
    return out, loss


# speedup vs baseline: 1.0118x; 1.0118x over previous
import functools

import jax
import jax.numpy as jnp
from jax import lax
from jax.experimental import pallas as pl
from jax.experimental.pallas import tpu as pltpu
from jax.experimental.pallas import tpu_sc as plsc

B, T, D, K = 8, 576, 256, 1024


CHUNKS = (576,)


BPS = 4


def _dist_body(boff, x_ref, cb_ref, idx_ref, loss_ref, cbn_s, caug_s):
    i = pl.program_id(0)

    @pl.when(i == 0)
    def _():
        cb = cb_ref[...]
        cbn_s[...] = cb * -2.0
        c_sq = jnp.sum(cb * cb, axis=1)
        caug_s[...] = jnp.broadcast_to(c_sq[None, :], (8, K))
        loss_ref[0, 0] = 0.0

    cbn = cbn_s[...]
    c_sq8 = caug_s[...]

    for bi in range(BPS):
        xi = x_ref[bi]
        x_sq = jnp.sum(xi * xi, axis=1)
        accv = jnp.full((8, K), jnp.inf, jnp.float32)
        acci = jnp.zeros((8, K), jnp.int32)
        cstart = 0
        for tb in CHUNKS:
            xi_c = xi[cstart:cstart + tb, :]
            dc = lax.dot_general(xi_c, cbn, (((1,), (1,)), ((), ())),
                                 preferred_element_type=jnp.float32)
            for g in range(tb // 8):
                t0 = cstart + g * 8
                blk = (c_sq8 + x_sq[t0:t0 + 8, None]) + dc[g * 8:(g + 1) * 8, :]
                upd = blk < accv
                accv = jnp.minimum(accv, blk)
                acci = jnp.where(upd, t0 // 8, acci)
            cstart += tb

        minv = jnp.min(accv, axis=0)
        tcand = acci * 8 + lax.broadcasted_iota(jnp.int32, (8, K), 0)
        amin = jnp.min(jnp.where(accv == minv[None, :], tcand, T), axis=0)
        idx_ref[bi, 0, :] = amin + (boff + i * BPS + bi) * T
        loss_ref[0, 0] += jnp.sum(minv)


def _distances_argmin(x, codebook, boff, nb):
    return pl.pallas_call(
        functools.partial(_dist_body, boff),
        grid=(nb // BPS,),
        in_specs=[
            pl.BlockSpec((BPS, T, D), lambda i: (i, 0, 0)),
            pl.BlockSpec((K, D), lambda i: (0, 0)),
        ],
        out_specs=[
            pl.BlockSpec((BPS, 1, K), lambda i: (i, 0, 0)),
            pl.BlockSpec(block_shape=(1, 1), index_map=lambda i: (0, 0),
                         memory_space=pltpu.SMEM),
        ],
        out_shape=[
            jax.ShapeDtypeStruct((nb, 1, K), jnp.int32),
            jax.ShapeDtypeStruct((1, 1), jnp.float32),
        ],
        scratch_shapes=[
            pltpu.VMEM((K, D), jnp.float32),
            pltpu.VMEM((8, K), jnp.float32),
        ],
    )(x, codebook)


_NC, _NS = 2, 16
_NW = _NC * _NS
_CHUNK = 64


def _gather_body(bpw, table_hbm, idx_hbm, out_hbm, idx_v, rows_v, sem_g0,
                 sem_g1, sem_g2, sem_g3, sem_w):
    wid = lax.axis_index("s") * _NC + lax.axis_index("c")
    base = wid * bpw
    pltpu.sync_copy(idx_hbm.at[pl.ds(base, bpw)], idx_v)
    sems = [sem_g0, sem_g1, sem_g2, sem_g3]
    gathers = [pltpu.async_copy(
        table_hbm.at[idx_v.at[pl.ds(c * _CHUNK, _CHUNK)]],
        rows_v.at[pl.ds(c * _CHUNK, _CHUNK)],
        sems[c]) for c in range(bpw // _CHUNK)]
    writes = []
    for c, g in enumerate(gathers):
        g.wait()
        writes.append(pltpu.async_copy(
            rows_v.at[pl.ds(c * _CHUNK, _CHUNK)],
            out_hbm.at[pl.ds(base + c * _CHUNK, _CHUNK)],
            sem_w))
    for w in writes:
        w.wait()


@functools.cache
def _gather_rows(nrows):
    bpw = nrows // _NW
    return functools.partial(
        pl.kernel,
        mesh=plsc.VectorSubcoreMesh(core_axis_name="c", subcore_axis_name="s"),
        out_type=jax.ShapeDtypeStruct((nrows, D), jnp.float32),
        scratch_types=[
            pltpu.VMEM((bpw,), jnp.int32),
            pltpu.VMEM((bpw, D), jnp.float32),
            pltpu.SemaphoreType.DMA,
            pltpu.SemaphoreType.DMA,
            pltpu.SemaphoreType.DMA,
            pltpu.SemaphoreType.DMA,
            pltpu.SemaphoreType.DMA,
        ],
    )(functools.partial(_gather_body, bpw))


def kernel(x, codebook):
    idx4, loss_sum = _distances_argmin(x, codebook, 0, B)
    table = x.reshape(B * T, D)
    out = _gather_rows(B * K)(table, idx4.reshape(B * K)).reshape(B, K, D)
    loss = loss_sum[0, 0] * (2.0 / (B * K * D))
    return out, loss
